# Initial kernel scaffold; baseline (speedup 1.0000x reference)
import jax
import jax.numpy as jnp
from jax.experimental import pallas as pl


def kernel(src, output, attention, W, b):
    raise NotImplementedError("write your pallas kernel here")


# speedup vs baseline: 1.9855x; 1.9855x over previous
import jax
import jax.numpy as jnp
from jax.experimental import pallas as pl
from jax.experimental.pallas import tpu as pltpu

_B, _T, _S, _V = 8, 32, 200, 100000
_TB = 16
_CV = 1408
_NC = 99968 // _CV
_TAIL = _V - _NC * _CV


def _body(b_ref, src_ref, out_in_ref, att_ref, w_ref, out_ref):
    o = out_in_ref[0]
    m = jnp.max(o, axis=1, keepdims=True)
    d = jnp.sum(o * w_ref[...], axis=1, keepdims=True)
    p = jax.nn.sigmoid(d + b_ref[0, 0])
    e = jnp.exp(o - m)
    z = jnp.sum(e, axis=1, keepdims=True)
    out_ref[0] = e * ((1.0 - p) / z)

    pa = p * att_ref[0]
    srcv = src_ref[0]

    def chunk(off, width):
        iota = jax.lax.broadcasted_iota(jnp.int32, (_S, width), 1) + off
        oh = (iota == srcv).astype(jnp.float32)
        cc = jnp.dot(pa, oh, preferred_element_type=jnp.float32)
        sl = pl.ds(off, width)
        out_ref[0, :, sl] = jnp.log(out_ref[0, :, sl] + cc + 1e-10)

    def loop(i, _):
        chunk(i * _CV, _CV)
        return 0

    jax.lax.fori_loop(0, _NC, loop, 0)
    chunk(_NC * _CV, _TAIL)


def _call(src, output, attention, W, b, interpret=False):
    b2 = b.reshape(1, 1)
    src3 = src.reshape(_B, _S, 1)
    out = pl.pallas_call(
        _body,
        grid=(_B, _T // _TB),
        in_specs=[
            pl.BlockSpec((1, 1), lambda i, j: (0, 0), memory_space=pltpu.SMEM),
            pl.BlockSpec((1, _S, 1), lambda i, j: (i, 0, 0)),
            pl.BlockSpec((1, _TB, _V), lambda i, j: (i, j, 0)),
            pl.BlockSpec((1, _TB, _S), lambda i, j: (i, j, 0)),
            pl.BlockSpec((1, _V), lambda i, j: (0, 0)),
        ],
        out_specs=pl.BlockSpec((1, _TB, _V), lambda i, j: (i, j, 0)),
        out_shape=jax.ShapeDtypeStruct((_B, _T, _V), jnp.float32),
        interpret=interpret,
    )(b2, src3, output, attention, W)
    return (out, attention)


def kernel(src, output, attention, W, b):
    return _call(src, output, attention, W, b)
